# baseline (device time: 77546 ns/iter reference)
import jax
import jax.numpy as jnp
from jax import lax
from jax.experimental import pallas as pl
from jax.experimental.pallas import tpu as pltpu

N_DEV = 16
N_STAGES = 4


def kernel(A, B):
    m, k = A.shape
    k2, n = B.shape

    def body(a_ref, b_ref, out_ref, comm_ref, send_sems, recv_sems):
        my = lax.axis_index("i")

        out_ref[...] = jnp.dot(
            a_ref[...], b_ref[...], preferred_element_type=jnp.float32
        )

        for st in range(N_STAGES):
            partner = lax.bitwise_xor(my, 2**st)
            rdma = pltpu.make_async_remote_copy(
                src_ref=out_ref,
                dst_ref=comm_ref.at[st],
                send_sem=send_sems.at[st],
                recv_sem=recv_sems.at[st],
                device_id=(partner,),
                device_id_type=pl.DeviceIdType.MESH,
            )
            rdma.start()
            rdma.wait()
            out_ref[...] = out_ref[...] + comm_ref[st]

        out_ref[...] = jnp.maximum(out_ref[...], 0.0)

    return pl.pallas_call(
        body,
        out_shape=jax.ShapeDtypeStruct((m, n), jnp.float32),
        in_specs=[
            pl.BlockSpec(memory_space=pltpu.VMEM),
            pl.BlockSpec(memory_space=pltpu.VMEM),
        ],
        out_specs=pl.BlockSpec(memory_space=pltpu.VMEM),
        scratch_shapes=[
            pltpu.VMEM((N_STAGES, m, n), jnp.float32),
            pltpu.SemaphoreType.DMA((N_STAGES,)),
            pltpu.SemaphoreType.DMA((N_STAGES,)),
        ],
    )(A, B)


# device time: 48128 ns/iter; 1.6112x vs baseline; 1.6112x over previous
import jax
import jax.numpy as jnp
from jax import lax
from jax.experimental import pallas as pl
from jax.experimental.pallas import tpu as pltpu

N_DEV = 16
RS_MASKS = (1, 4, 2, 8)
AG_MASKS = (8, 2, 4, 1)


def kernel(A, B):
    m, k = A.shape
    k2, n = B.shape

    def body(a_ref, b_ref, out_ref, comm_ref, send_sems, recv_sems):
        my = lax.axis_index("i")

        out_ref[...] = jnp.dot(
            a_ref[...], b_ref[...], preferred_element_type=jnp.float32
        )

        off = jnp.int32(0)
        sz = m
        comm_off = 0
        for s, mask in enumerate(RS_MASKS):
            half = sz // 2
            partner = lax.bitwise_xor(my, mask)
            bit = lax.bitwise_and(my, mask) > 0
            send_off = off + jnp.where(bit, 0, half)
            keep_off = off + jnp.where(bit, half, 0)
            rdma = pltpu.make_async_remote_copy(
                src_ref=out_ref.at[pl.ds(send_off, half), :],
                dst_ref=comm_ref.at[pl.ds(comm_off, half), :],
                send_sem=send_sems.at[s],
                recv_sem=recv_sems.at[s],
                device_id=(partner,),
                device_id_type=pl.DeviceIdType.MESH,
            )
            rdma.start()
            rdma.wait()
            out_ref[pl.ds(keep_off, half), :] = (
                out_ref[pl.ds(keep_off, half), :]
                + comm_ref[pl.ds(comm_off, half), :]
            )
            off = keep_off
            sz = half
            comm_off += half

        for s, mask in enumerate(AG_MASKS):
            partner = lax.bitwise_xor(my, mask)
            bit = lax.bitwise_and(my, mask) > 0
            parent_off = off - jnp.where(bit, sz, 0)
            rdma = pltpu.make_async_remote_copy(
                src_ref=out_ref.at[pl.ds(off, sz), :],
                dst_ref=out_ref.at[pl.ds(off, sz), :],
                send_sem=send_sems.at[4 + s],
                recv_sem=recv_sems.at[4 + s],
                device_id=(partner,),
                device_id_type=pl.DeviceIdType.MESH,
            )
            rdma.start()
            rdma.wait()
            off = parent_off
            sz = sz * 2

        out_ref[...] = jnp.maximum(out_ref[...], 0.0)

    return pl.pallas_call(
        body,
        out_shape=jax.ShapeDtypeStruct((m, n), jnp.float32),
        in_specs=[
            pl.BlockSpec(memory_space=pltpu.VMEM),
            pl.BlockSpec(memory_space=pltpu.VMEM),
        ],
        out_specs=pl.BlockSpec(memory_space=pltpu.VMEM),
        scratch_shapes=[
            pltpu.VMEM((m, n), jnp.float32),
            pltpu.SemaphoreType.DMA((8,)),
            pltpu.SemaphoreType.DMA((8,)),
        ],
    )(A, B)


# device time: 39957 ns/iter; 1.9407x vs baseline; 1.2045x over previous
import jax
import jax.numpy as jnp
from jax import lax
from jax.experimental import pallas as pl
from jax.experimental.pallas import tpu as pltpu

N_DEV = 16
RS_MASKS = ((1, 4, 2, 8), (4, 1, 8, 2))
AG_MASKS = ((8, 2, 4, 1), (2, 8, 1, 4))


def kernel(A, B):
    m, k = A.shape
    k2, n = B.shape
    ncol = n // 2

    def body(a_ref, b_ref, out_ref, comm_ref, send_sems, recv_sems):
        my = lax.axis_index("i")

        out_ref[...] = jnp.dot(
            a_ref[...], b_ref[...], preferred_element_type=jnp.float32
        )

        off = [jnp.int32(0), jnp.int32(0)]
        sz = m
        comm_off = 0
        for s in range(4):
            half = sz // 2
            rdmas = []
            keep_offs = []
            for hv in range(2):
                mask = RS_MASKS[hv][s]
                partner = lax.bitwise_xor(my, mask)
                bit = lax.bitwise_and(my, mask) > 0
                send_off = off[hv] + jnp.where(bit, 0, half)
                keep_offs.append(off[hv] + jnp.where(bit, half, 0))
                rdma = pltpu.make_async_remote_copy(
                    src_ref=out_ref.at[
                        pl.ds(send_off, half), pl.ds(hv * ncol, ncol)
                    ],
                    dst_ref=comm_ref.at[
                        pl.ds(comm_off, half), pl.ds(hv * ncol, ncol)
                    ],
                    send_sem=send_sems.at[hv, s],
                    recv_sem=recv_sems.at[hv, s],
                    device_id=(partner,),
                    device_id_type=pl.DeviceIdType.MESH,
                )
                rdma.start()
                rdmas.append(rdma)
            for hv in range(2):
                rdmas[hv].wait()
                out_ref[pl.ds(keep_offs[hv], half), pl.ds(hv * ncol, ncol)] = (
                    out_ref[pl.ds(keep_offs[hv], half), pl.ds(hv * ncol, ncol)]
                    + comm_ref[pl.ds(comm_off, half), pl.ds(hv * ncol, ncol)]
                )
                off[hv] = keep_offs[hv]
            sz = half
            comm_off += half

        for s in range(4):
            rdmas = []
            parent_offs = []
            for hv in range(2):
                mask = AG_MASKS[hv][s]
                partner = lax.bitwise_xor(my, mask)
                bit = lax.bitwise_and(my, mask) > 0
                parent_offs.append(off[hv] - jnp.where(bit, sz, 0))
                rdma = pltpu.make_async_remote_copy(
                    src_ref=out_ref.at[
                        pl.ds(off[hv], sz), pl.ds(hv * ncol, ncol)
                    ],
                    dst_ref=out_ref.at[
                        pl.ds(off[hv], sz), pl.ds(hv * ncol, ncol)
                    ],
                    send_sem=send_sems.at[hv, 4 + s],
                    recv_sem=recv_sems.at[hv, 4 + s],
                    device_id=(partner,),
                    device_id_type=pl.DeviceIdType.MESH,
                )
                rdma.start()
                rdmas.append(rdma)
            for hv in range(2):
                rdmas[hv].wait()
                off[hv] = parent_offs[hv]
            sz = sz * 2

        out_ref[...] = jnp.maximum(out_ref[...], 0.0)

    return pl.pallas_call(
        body,
        out_shape=jax.ShapeDtypeStruct((m, n), jnp.float32),
        in_specs=[
            pl.BlockSpec(memory_space=pltpu.VMEM),
            pl.BlockSpec(memory_space=pltpu.VMEM),
        ],
        out_specs=pl.BlockSpec(memory_space=pltpu.VMEM),
        scratch_shapes=[
            pltpu.VMEM((m, n), jnp.float32),
            pltpu.SemaphoreType.DMA((2, 8)),
            pltpu.SemaphoreType.DMA((2, 8)),
        ],
    )(A, B)


# device time: 33030 ns/iter; 2.3477x vs baseline; 1.2097x over previous
import jax
import jax.numpy as jnp
from jax import lax
from jax.experimental import pallas as pl
from jax.experimental.pallas import tpu as pltpu

N_DEV = 16
RS_MASKS = ((1, 4, 2, 8), (4, 1, 8, 2))
AG_MASKS = ((8, 2, 4, 1), (2, 8, 1, 4))
RS_COMM_OFFS = (0, 256, 384, 448)


def kernel(A, B):
    m, k = A.shape
    k2, n = B.shape
    ncol = n // 2

    def body(a_ref, b_ref, out_ref, comm_ref, send_sems, recv_sems):
        my = lax.axis_index("i")

        barrier = pltpu.get_barrier_semaphore()
        for mask in (1, 2, 4, 8):
            pl.semaphore_signal(
                barrier,
                inc=1,
                device_id=(lax.bitwise_xor(my, mask),),
                device_id_type=pl.DeviceIdType.MESH,
            )
        pl.semaphore_wait(barrier, 4)

        out_ref[...] = jnp.dot(
            a_ref[...], b_ref[...], preferred_element_type=jnp.float32
        )

        pending_sends = []

        def rs_start(hv, s, off):
            sz = m >> s
            half = sz // 2
            mask = RS_MASKS[hv][s]
            partner = lax.bitwise_xor(my, mask)
            bit = lax.bitwise_and(my, mask) > 0
            send_off = off + jnp.where(bit, 0, half)
            keep_off = off + jnp.where(bit, half, 0)
            rdma = pltpu.make_async_remote_copy(
                src_ref=out_ref.at[
                    pl.ds(send_off, half), pl.ds(hv * ncol, ncol)
                ],
                dst_ref=comm_ref.at[
                    pl.ds(RS_COMM_OFFS[s], half), pl.ds(hv * ncol, ncol)
                ],
                send_sem=send_sems.at[hv, s],
                recv_sem=recv_sems.at[hv, s],
                device_id=(partner,),
                device_id_type=pl.DeviceIdType.MESH,
            )
            rdma.start()
            pending_sends.append(rdma)
            return rdma, keep_off

        def ag_start(hv, s, off):
            sz = (m >> 4) << s
            mask = AG_MASKS[hv][s]
            partner = lax.bitwise_xor(my, mask)
            bit = lax.bitwise_and(my, mask) > 0
            parent_off = off - jnp.where(bit, sz, 0)
            rdma = pltpu.make_async_remote_copy(
                src_ref=out_ref.at[pl.ds(off, sz), pl.ds(hv * ncol, ncol)],
                dst_ref=out_ref.at[pl.ds(off, sz), pl.ds(hv * ncol, ncol)],
                send_sem=send_sems.at[hv, 4 + s],
                recv_sem=recv_sems.at[hv, 4 + s],
                device_id=(partner,),
                device_id_type=pl.DeviceIdType.MESH,
            )
            rdma.start()
            pending_sends.append(rdma)
            return rdma, parent_off

        off = [jnp.int32(0), jnp.int32(0)]
        rdma = [None, None]
        for hv in range(2):
            rdma[hv], off[hv] = rs_start(hv, 0, off[hv])

        for s in range(4):
            half = m >> (s + 1)
            for hv in range(2):
                rdma[hv].wait_recv()
                out_ref[pl.ds(off[hv], half), pl.ds(hv * ncol, ncol)] = (
                    out_ref[pl.ds(off[hv], half), pl.ds(hv * ncol, ncol)]
                    + comm_ref[
                        pl.ds(RS_COMM_OFFS[s], half), pl.ds(hv * ncol, ncol)
                    ]
                )
                if s < 3:
                    rdma[hv], off[hv] = rs_start(hv, s + 1, off[hv])
                else:
                    rdma[hv], off[hv] = ag_start(hv, 0, off[hv])

        for s in range(4):
            for hv in range(2):
                rdma[hv].wait_recv()
                if s < 3:
                    rdma[hv], off[hv] = ag_start(hv, s + 1, off[hv])

        for r in pending_sends:
            r.wait_send()

        out_ref[...] = jnp.maximum(out_ref[...], 0.0)

    return pl.pallas_call(
        body,
        out_shape=jax.ShapeDtypeStruct((m, n), jnp.float32),
        in_specs=[
            pl.BlockSpec(memory_space=pltpu.VMEM),
            pl.BlockSpec(memory_space=pltpu.VMEM),
        ],
        out_specs=pl.BlockSpec(memory_space=pltpu.VMEM),
        scratch_shapes=[
            pltpu.VMEM((m, n), jnp.float32),
            pltpu.SemaphoreType.DMA((2, 8)),
            pltpu.SemaphoreType.DMA((2, 8)),
        ],
        compiler_params=pltpu.CompilerParams(collective_id=0),
    )(A, B)


# device time: 26734 ns/iter; 2.9007x vs baseline; 1.2355x over previous
import jax
import jax.numpy as jnp
from jax import lax
from jax.experimental import pallas as pl
from jax.experimental.pallas import tpu as pltpu

N_DEV = 16
RS_ORDER = (("Y", "Z1", "X", "Z2"), ("Z1", "Y", "Z2", "X"))
AG_ORDER = tuple(tuple(reversed(o)) for o in RS_ORDER)
RS_COMM_OFFS = (0, 256, 384, 448)


def _partner_and_bit(my, cls):
    if cls == "X":
        return lax.bitwise_xor(my, 1), lax.bitwise_and(my, 1) > 0
    if cls == "Y":
        p = lax.bitwise_and(my, 3)
        return my + 3 - 2 * p, lax.bitwise_and(my, 2) > 0
    if cls == "Z1":
        return lax.bitwise_xor(my, 4), lax.bitwise_and(my, 4) > 0
    assert cls == "Z2"
    return lax.bitwise_xor(my, 8), lax.bitwise_and(my, 8) > 0


def kernel(A, B):
    m, k = A.shape
    k2, n = B.shape
    ncol = n // 2

    def body(
        a_ref, b_ref, out_ref, comm_ref, sbuf_ref, gbuf_ref, send_sems, recv_sems
    ):
        my = lax.axis_index("i")

        barrier = pltpu.get_barrier_semaphore()
        for cls in ("X", "Y", "Z1", "Z2"):
            partner, _ = _partner_and_bit(my, cls)
            pl.semaphore_signal(
                barrier,
                inc=1,
                device_id=(partner,),
                device_id_type=pl.DeviceIdType.MESH,
            )

        out_ref[...] = jnp.dot(
            a_ref[...], b_ref[...], preferred_element_type=jnp.float32
        )

        pl.semaphore_wait(barrier, 4)

        pending_sends = []

        def rs_start(hv, s, off):
            half = m >> (s + 1)
            partner, bit = _partner_and_bit(my, RS_ORDER[hv][s])
            send_off = off + jnp.where(bit, 0, half)
            keep_off = off + jnp.where(bit, half, 0)
            rows = pl.ds(RS_COMM_OFFS[s], half)
            cols = pl.ds(hv * ncol, ncol)
            sbuf_ref[rows, cols] = out_ref[pl.ds(send_off, half), cols].astype(
                jnp.bfloat16
            )
            rdma = pltpu.make_async_remote_copy(
                src_ref=sbuf_ref.at[rows, cols],
                dst_ref=comm_ref.at[rows, cols],
                send_sem=send_sems.at[hv, s],
                recv_sem=recv_sems.at[hv, s],
                device_id=(partner,),
                device_id_type=pl.DeviceIdType.MESH,
            )
            rdma.start()
            pending_sends.append(rdma)
            return rdma, keep_off

        def ag_start(hv, s, off):
            sz = (m >> 4) << s
            partner, bit = _partner_and_bit(my, AG_ORDER[hv][s])
            parent_off = off - jnp.where(bit, sz, 0)
            rdma = pltpu.make_async_remote_copy(
                src_ref=gbuf_ref.at[pl.ds(off, sz), pl.ds(hv * ncol, ncol)],
                dst_ref=gbuf_ref.at[pl.ds(off, sz), pl.ds(hv * ncol, ncol)],
                send_sem=send_sems.at[hv, 4 + s],
                recv_sem=recv_sems.at[hv, 4 + s],
                device_id=(partner,),
                device_id_type=pl.DeviceIdType.MESH,
            )
            rdma.start()
            pending_sends.append(rdma)
            return rdma, parent_off

        off = [jnp.int32(0), jnp.int32(0)]
        rdma = [None, None]
        for hv in range(2):
            rdma[hv], off[hv] = rs_start(hv, 0, off[hv])

        for s in range(4):
            half = m >> (s + 1)
            for hv in range(2):
                cols = pl.ds(hv * ncol, ncol)
                rdma[hv].wait_recv()
                out_ref[pl.ds(off[hv], half), cols] = (
                    out_ref[pl.ds(off[hv], half), cols]
                    + comm_ref[pl.ds(RS_COMM_OFFS[s], half), cols].astype(
                        jnp.float32
                    )
                )
                if s < 3:
                    rdma[hv], off[hv] = rs_start(hv, s + 1, off[hv])
                else:
                    blk = pl.ds(off[hv], m >> 4)
                    gbuf_ref[blk, cols] = out_ref[blk, cols].astype(
                        jnp.bfloat16
                    )
                    rdma[hv], off[hv] = ag_start(hv, 0, off[hv])

        for s in range(4):
            for hv in range(2):
                rdma[hv].wait_recv()
                if s < 3:
                    rdma[hv], off[hv] = ag_start(hv, s + 1, off[hv])

        out_ref[...] = jnp.maximum(gbuf_ref[...].astype(jnp.float32), 0.0)

        for r in pending_sends:
            r.wait_send()

    return pl.pallas_call(
        body,
        out_shape=jax.ShapeDtypeStruct((m, n), jnp.float32),
        in_specs=[
            pl.BlockSpec(memory_space=pltpu.VMEM),
            pl.BlockSpec(memory_space=pltpu.VMEM),
        ],
        out_specs=pl.BlockSpec(memory_space=pltpu.VMEM),
        scratch_shapes=[
            pltpu.VMEM((m, n), jnp.bfloat16),
            pltpu.VMEM((m, n), jnp.bfloat16),
            pltpu.VMEM((m, n), jnp.bfloat16),
            pltpu.SemaphoreType.DMA((2, 8)),
            pltpu.SemaphoreType.DMA((2, 8)),
        ],
        compiler_params=pltpu.CompilerParams(collective_id=0),
    )(A, B)


# device time: 26678 ns/iter; 2.9067x vs baseline; 1.0021x over previous
import jax
import jax.numpy as jnp
from jax import lax
from jax.experimental import pallas as pl
from jax.experimental.pallas import tpu as pltpu

N_DEV = 16
RS_ORDER = (("Y", "Z1", "X", "Z2"), ("Z1", "Y", "Z2", "X"))
AG_ORDER = tuple(tuple(reversed(o)) for o in RS_ORDER)
RS_COMM_OFFS = (0, 256, 384, 448)


def _partner_and_bit(my, cls):
    if cls == "X":
        return lax.bitwise_xor(my, 1), lax.bitwise_and(my, 1) > 0
    if cls == "Y":
        p = lax.bitwise_and(my, 3)
        return my + 3 - 2 * p, lax.bitwise_and(my, 2) > 0
    if cls == "Z1":
        return lax.bitwise_xor(my, 4), lax.bitwise_and(my, 4) > 0
    assert cls == "Z2"
    return lax.bitwise_xor(my, 8), lax.bitwise_and(my, 8) > 0


def kernel(A, B):
    m, k = A.shape
    k2, n = B.shape
    ncol = n // 2

    def body(
        a_ref, b_ref, out_ref, comm_ref, sbuf_ref, gbuf_ref, send_sems, recv_sems
    ):
        my = lax.axis_index("i")

        barrier = pltpu.get_barrier_semaphore()
        for cls in ("X", "Y", "Z1", "Z2"):
            partner, _ = _partner_and_bit(my, cls)
            pl.semaphore_signal(
                barrier,
                inc=1,
                device_id=(partner,),
                device_id_type=pl.DeviceIdType.MESH,
            )

        half0 = m // 2
        for hv in range(2):
            _, bit = _partner_and_bit(my, RS_ORDER[hv][0])
            send_off = jnp.where(bit, 0, half0)
            out_ref[pl.ds(send_off, half0), pl.ds(hv * ncol, ncol)] = jnp.dot(
                a_ref[pl.ds(send_off, half0), :],
                b_ref[:, hv * ncol : (hv + 1) * ncol],
                preferred_element_type=jnp.float32,
            )

        pl.semaphore_wait(barrier, 4)

        pending_sends = []

        def rs_start(hv, s, off):
            half = m >> (s + 1)
            partner, bit = _partner_and_bit(my, RS_ORDER[hv][s])
            send_off = off + jnp.where(bit, 0, half)
            keep_off = off + jnp.where(bit, half, 0)
            rows = pl.ds(RS_COMM_OFFS[s], half)
            cols = pl.ds(hv * ncol, ncol)
            sbuf_ref[rows, cols] = out_ref[pl.ds(send_off, half), cols].astype(
                jnp.bfloat16
            )
            rdma = pltpu.make_async_remote_copy(
                src_ref=sbuf_ref.at[rows, cols],
                dst_ref=comm_ref.at[rows, cols],
                send_sem=send_sems.at[hv, s],
                recv_sem=recv_sems.at[hv, s],
                device_id=(partner,),
                device_id_type=pl.DeviceIdType.MESH,
            )
            rdma.start()
            pending_sends.append(rdma)
            return rdma, keep_off

        def ag_start(hv, s, off):
            sz = (m >> 4) << s
            partner, bit = _partner_and_bit(my, AG_ORDER[hv][s])
            parent_off = off - jnp.where(bit, sz, 0)
            rdma = pltpu.make_async_remote_copy(
                src_ref=gbuf_ref.at[pl.ds(off, sz), pl.ds(hv * ncol, ncol)],
                dst_ref=gbuf_ref.at[pl.ds(off, sz), pl.ds(hv * ncol, ncol)],
                send_sem=send_sems.at[hv, 4 + s],
                recv_sem=recv_sems.at[hv, 4 + s],
                device_id=(partner,),
                device_id_type=pl.DeviceIdType.MESH,
            )
            rdma.start()
            pending_sends.append(rdma)
            return rdma, parent_off

        off = [jnp.int32(0), jnp.int32(0)]
        rdma = [None, None]
        for hv in range(2):
            rdma[hv], off[hv] = rs_start(hv, 0, off[hv])

        for hv in range(2):
            _, bit = _partner_and_bit(my, RS_ORDER[hv][0])
            keep_off = jnp.where(bit, half0, 0)
            out_ref[pl.ds(keep_off, half0), pl.ds(hv * ncol, ncol)] = jnp.dot(
                a_ref[pl.ds(keep_off, half0), :],
                b_ref[:, hv * ncol : (hv + 1) * ncol],
                preferred_element_type=jnp.float32,
            )

        for s in range(4):
            half = m >> (s + 1)
            for hv in range(2):
                cols = pl.ds(hv * ncol, ncol)
                rdma[hv].wait_recv()
                out_ref[pl.ds(off[hv], half), cols] = (
                    out_ref[pl.ds(off[hv], half), cols]
                    + comm_ref[pl.ds(RS_COMM_OFFS[s], half), cols].astype(
                        jnp.float32
                    )
                )
                if s < 3:
                    rdma[hv], off[hv] = rs_start(hv, s + 1, off[hv])
                else:
                    blk = pl.ds(off[hv], m >> 4)
                    gbuf_ref[blk, cols] = out_ref[blk, cols].astype(
                        jnp.bfloat16
                    )
                    rdma[hv], off[hv] = ag_start(hv, 0, off[hv])

        for s in range(4):
            for hv in range(2):
                rdma[hv].wait_recv()
                if s < 3:
                    rdma[hv], off[hv] = ag_start(hv, s + 1, off[hv])

        out_ref[...] = jnp.maximum(gbuf_ref[...].astype(jnp.float32), 0.0)

        for r in pending_sends:
            r.wait_send()

    return pl.pallas_call(
        body,
        out_shape=jax.ShapeDtypeStruct((m, n), jnp.float32),
        in_specs=[
            pl.BlockSpec(memory_space=pltpu.VMEM),
            pl.BlockSpec(memory_space=pltpu.VMEM),
        ],
        out_specs=pl.BlockSpec(memory_space=pltpu.VMEM),
        scratch_shapes=[
            pltpu.VMEM((m, n), jnp.bfloat16),
            pltpu.VMEM((m, n), jnp.bfloat16),
            pltpu.VMEM((m, n), jnp.bfloat16),
            pltpu.SemaphoreType.DMA((2, 8)),
            pltpu.SemaphoreType.DMA((2, 8)),
        ],
        compiler_params=pltpu.CompilerParams(collective_id=0),
    )(A, B)


# device time: 25312 ns/iter; 3.0636x vs baseline; 1.0540x over previous
import jax
import jax.numpy as jnp
from jax import lax
from jax.experimental import pallas as pl
from jax.experimental.pallas import tpu as pltpu

N_DEV = 16
STAGES = (("Y", "Z1", "X", "Z2", "Y"), ("Z1", "Y", "Z2", "X", "Z1"))


def _partner_and_bit(my, cls):
    if cls == "X":
        return lax.bitwise_xor(my, 1), lax.bitwise_and(my, 1) > 0
    if cls == "Y":
        p = lax.bitwise_and(my, 3)
        return my + 3 - 2 * p, lax.bitwise_and(my, 2) > 0
    if cls == "Z1":
        return lax.bitwise_xor(my, 4), lax.bitwise_and(my, 4) > 0
    assert cls == "Z2"
    return lax.bitwise_xor(my, 8), lax.bitwise_and(my, 8) > 0


def kernel(A, B):
    m, k = A.shape
    k2, n = B.shape
    ncol = n // 2
    blk = m // 2

    def body(
        a_ref, b_ref, out_ref, comm_ref, sbuf_ref, gbuf_ref, send_sems, recv_sems
    ):
        my = lax.axis_index("i")

        barrier = pltpu.get_barrier_semaphore()
        for cls in ("X", "Y", "Z1", "Z2"):
            partner, _ = _partner_and_bit(my, cls)
            pl.semaphore_signal(
                barrier,
                inc=1,
                device_id=(partner,),
                device_id_type=pl.DeviceIdType.MESH,
            )

        bit0 = [None, None]
        for hv in range(2):
            _, bit0[hv] = _partner_and_bit(my, STAGES[hv][0])
            send_off = jnp.where(bit0[hv], 0, blk)
            out_ref[pl.ds(send_off, blk), pl.ds(hv * ncol, ncol)] = jnp.dot(
                a_ref[pl.ds(send_off, blk), :],
                b_ref[:, hv * ncol : (hv + 1) * ncol],
                preferred_element_type=jnp.float32,
            )

        pl.semaphore_wait(barrier, 4)

        pending_sends = []

        def start_stage(hv, s, off):
            partner, _ = _partner_and_bit(my, STAGES[hv][s])
            rows = pl.ds(s * blk, blk)
            cols = pl.ds(hv * ncol, ncol)
            sbuf_ref[rows, cols] = out_ref[pl.ds(off, blk), cols].astype(
                jnp.bfloat16
            )
            rdma = pltpu.make_async_remote_copy(
                src_ref=sbuf_ref.at[rows, cols],
                dst_ref=comm_ref.at[rows, cols],
                send_sem=send_sems.at[hv, s],
                recv_sem=recv_sems.at[hv, s],
                device_id=(partner,),
                device_id_type=pl.DeviceIdType.MESH,
            )
            rdma.start()
            pending_sends.append(rdma)
            return rdma

        off = [None, None]
        rdma = [None, None]
        for hv in range(2):
            send_off = jnp.where(bit0[hv], 0, blk)
            off[hv] = jnp.where(bit0[hv], blk, 0)
            rdma[hv] = start_stage(hv, 0, send_off)

        for hv in range(2):
            out_ref[pl.ds(off[hv], blk), pl.ds(hv * ncol, ncol)] = jnp.dot(
                a_ref[pl.ds(off[hv], blk), :],
                b_ref[:, hv * ncol : (hv + 1) * ncol],
                preferred_element_type=jnp.float32,
            )

        for s in range(4):
            for hv in range(2):
                cols = pl.ds(hv * ncol, ncol)
                rdma[hv].wait_recv()
                out_ref[pl.ds(off[hv], blk), cols] = (
                    out_ref[pl.ds(off[hv], blk), cols]
                    + comm_ref[pl.ds(s * blk, blk), cols].astype(jnp.float32)
                )
                if s < 3:
                    rdma[hv] = start_stage(hv, s + 1, off[hv])
                else:
                    partner, _ = _partner_and_bit(my, STAGES[hv][4])
                    gbuf_ref[pl.ds(off[hv], blk), cols] = out_ref[
                        pl.ds(off[hv], blk), cols
                    ].astype(jnp.bfloat16)
                    r = pltpu.make_async_remote_copy(
                        src_ref=gbuf_ref.at[pl.ds(off[hv], blk), cols],
                        dst_ref=gbuf_ref.at[pl.ds(off[hv], blk), cols],
                        send_sem=send_sems.at[hv, 4],
                        recv_sem=recv_sems.at[hv, 4],
                        device_id=(partner,),
                        device_id_type=pl.DeviceIdType.MESH,
                    )
                    r.start()
                    pending_sends.append(r)
                    rdma[hv] = r

        for hv in range(2):
            rdma[hv].wait_recv()

        out_ref[...] = jnp.maximum(gbuf_ref[...].astype(jnp.float32), 0.0)

        for r in pending_sends:
            r.wait_send()

    return pl.pallas_call(
        body,
        out_shape=jax.ShapeDtypeStruct((m, n), jnp.float32),
        in_specs=[
            pl.BlockSpec(memory_space=pltpu.VMEM),
            pl.BlockSpec(memory_space=pltpu.VMEM),
        ],
        out_specs=pl.BlockSpec(memory_space=pltpu.VMEM),
        scratch_shapes=[
            pltpu.VMEM((4 * (m // 2), n), jnp.bfloat16),
            pltpu.VMEM((4 * (m // 2), n), jnp.bfloat16),
            pltpu.VMEM((m, n), jnp.bfloat16),
            pltpu.SemaphoreType.DMA((2, 5)),
            pltpu.SemaphoreType.DMA((2, 5)),
        ],
        compiler_params=pltpu.CompilerParams(collective_id=0),
    )(A, B)


# device time: 25041 ns/iter; 3.0968x vs baseline; 1.0108x over previous
import jax
import jax.numpy as jnp
from jax import lax
from jax.experimental import pallas as pl
from jax.experimental.pallas import tpu as pltpu

N_DEV = 16
STAGES = (("Y", "Z1", "X", "Z2", "Y"), ("Z1", "Y", "Z2", "X", "Z1"))


def _partner_and_bit(my, cls):
    if cls == "X":
        return lax.bitwise_xor(my, 1), lax.bitwise_and(my, 1) > 0
    if cls == "Y":
        p = lax.bitwise_and(my, 3)
        return my + 3 - 2 * p, lax.bitwise_and(my, 2) > 0
    if cls == "Z1":
        return lax.bitwise_xor(my, 4), lax.bitwise_and(my, 4) > 0
    assert cls == "Z2"
    return lax.bitwise_xor(my, 8), lax.bitwise_and(my, 8) > 0


def kernel(A, B):
    m, k = A.shape
    k2, n = B.shape
    ncol = n // 2
    blk = m // 2

    def body(
        a_ref, b_ref, out_ref, comm_ref, sbuf_ref, gbuf_ref, send_sems, recv_sems
    ):
        my = lax.axis_index("i")

        barrier = pltpu.get_barrier_semaphore()
        for cls in ("X", "Y", "Z1", "Z2"):
            partner, _ = _partner_and_bit(my, cls)
            pl.semaphore_signal(
                barrier,
                inc=1,
                device_id=(partner,),
                device_id_type=pl.DeviceIdType.MESH,
            )

        bit0 = [None, None]
        for hv in range(2):
            _, bit0[hv] = _partner_and_bit(my, STAGES[hv][0])
            send_off = jnp.where(bit0[hv], 0, blk)
            sbuf_ref[pl.ds(0, blk), pl.ds(hv * ncol, ncol)] = jnp.dot(
                a_ref[pl.ds(send_off, blk), :],
                b_ref[:, hv * ncol : (hv + 1) * ncol],
                preferred_element_type=jnp.float32,
            ).astype(jnp.bfloat16)

        pl.semaphore_wait(barrier, 4)

        pending_sends = []

        def start_stage(hv, s):
            partner, _ = _partner_and_bit(my, STAGES[hv][s])
            rows = pl.ds(s * blk, blk)
            cols = pl.ds(hv * ncol, ncol)
            rdma = pltpu.make_async_remote_copy(
                src_ref=sbuf_ref.at[rows, cols],
                dst_ref=comm_ref.at[rows, cols],
                send_sem=send_sems.at[hv, s],
                recv_sem=recv_sems.at[hv, s],
                device_id=(partner,),
                device_id_type=pl.DeviceIdType.MESH,
            )
            rdma.start()
            pending_sends.append(rdma)
            return rdma

        off = [None, None]
        rdma = [None, None]
        for hv in range(2):
            off[hv] = jnp.where(bit0[hv], blk, 0)
            rdma[hv] = start_stage(hv, 0)

        for hv in range(2):
            out_ref[pl.ds(off[hv], blk), pl.ds(hv * ncol, ncol)] = jnp.dot(
                a_ref[pl.ds(off[hv], blk), :],
                b_ref[:, hv * ncol : (hv + 1) * ncol],
                preferred_element_type=jnp.float32,
            )

        for s in range(4):
            for hv in range(2):
                cols = pl.ds(hv * ncol, ncol)
                rdma[hv].wait_recv()
                t = (
                    out_ref[pl.ds(off[hv], blk), cols]
                    + comm_ref[pl.ds(s * blk, blk), cols].astype(jnp.float32)
                )
                if s < 3:
                    out_ref[pl.ds(off[hv], blk), cols] = t
                    sbuf_ref[pl.ds((s + 1) * blk, blk), cols] = t.astype(
                        jnp.bfloat16
                    )
                    rdma[hv] = start_stage(hv, s + 1)
                else:
                    partner, _ = _partner_and_bit(my, STAGES[hv][4])
                    gbuf_ref[pl.ds(off[hv], blk), cols] = t.astype(
                        jnp.bfloat16
                    )
                    r = pltpu.make_async_remote_copy(
                        src_ref=gbuf_ref.at[pl.ds(off[hv], blk), cols],
                        dst_ref=gbuf_ref.at[pl.ds(off[hv], blk), cols],
                        send_sem=send_sems.at[hv, 4],
                        recv_sem=recv_sems.at[hv, 4],
                        device_id=(partner,),
                        device_id_type=pl.DeviceIdType.MESH,
                    )
                    r.start()
                    pending_sends.append(r)
                    rdma[hv] = r

        for hv in range(2):
            rdma[hv].wait_recv()

        out_ref[...] = jnp.maximum(gbuf_ref[...].astype(jnp.float32), 0.0)

        for r in pending_sends:
            r.wait_send()

    return pl.pallas_call(
        body,
        out_shape=jax.ShapeDtypeStruct((m, n), jnp.float32),
        in_specs=[
            pl.BlockSpec(memory_space=pltpu.VMEM),
            pl.BlockSpec(memory_space=pltpu.VMEM),
        ],
        out_specs=pl.BlockSpec(memory_space=pltpu.VMEM),
        scratch_shapes=[
            pltpu.VMEM((4 * (m // 2), n), jnp.bfloat16),
            pltpu.VMEM((4 * (m // 2), n), jnp.bfloat16),
            pltpu.VMEM((m, n), jnp.bfloat16),
            pltpu.SemaphoreType.DMA((2, 5)),
            pltpu.SemaphoreType.DMA((2, 5)),
        ],
        compiler_params=pltpu.CompilerParams(collective_id=0),
    )(A, B)


# device time: 22715 ns/iter; 3.4139x vs baseline; 1.1024x over previous
import jax
import jax.numpy as jnp
from jax import lax
from jax.experimental import pallas as pl
from jax.experimental.pallas import tpu as pltpu

N_DEV = 16
STAGES = (("Y", "Z1", "X", "Z2", "Y"), ("Z1", "Y", "Z2", "X", "Z1"))
N_CHUNKS = 2


def _partner_and_bit(my, cls):
    if cls == "X":
        return lax.bitwise_xor(my, 1), lax.bitwise_and(my, 1) > 0
    if cls == "Y":
        p = lax.bitwise_and(my, 3)
        return my + 3 - 2 * p, lax.bitwise_and(my, 2) > 0
    if cls == "Z1":
        return lax.bitwise_xor(my, 4), lax.bitwise_and(my, 4) > 0
    assert cls == "Z2"
    return lax.bitwise_xor(my, 8), lax.bitwise_and(my, 8) > 0


def kernel(A, B):
    m, k = A.shape
    k2, n = B.shape
    ncol = n // 2
    blk = m // 2
    cblk = blk // N_CHUNKS

    def body(
        a_ref, b_ref, out_ref, comm_ref, sbuf_ref, gbuf_ref, send_sems, recv_sems
    ):
        my = lax.axis_index("i")

        barrier = pltpu.get_barrier_semaphore()
        for cls in ("X", "Y", "Z1", "Z2"):
            partner, _ = _partner_and_bit(my, cls)
            pl.semaphore_signal(
                barrier,
                inc=1,
                device_id=(partner,),
                device_id_type=pl.DeviceIdType.MESH,
            )

        pending_sends = []

        def start_stage(hv, s, c):
            partner, _ = _partner_and_bit(my, STAGES[hv][s])
            rows = pl.ds(s * blk + c * cblk, cblk)
            cols = pl.ds(hv * ncol, ncol)
            rdma = pltpu.make_async_remote_copy(
                src_ref=sbuf_ref.at[rows, cols],
                dst_ref=comm_ref.at[rows, cols],
                send_sem=send_sems.at[hv, s, c],
                recv_sem=recv_sems.at[hv, s, c],
                device_id=(partner,),
                device_id_type=pl.DeviceIdType.MESH,
            )
            rdma.start()
            pending_sends.append(rdma)
            return rdma

        bit0 = [None, None]
        send_off = [None, None]
        off = [None, None]
        for hv in range(2):
            _, bit0[hv] = _partner_and_bit(my, STAGES[hv][0])
            send_off[hv] = jnp.where(bit0[hv], 0, blk)
            off[hv] = jnp.where(bit0[hv], blk, 0)

        rdma = [[None] * N_CHUNKS, [None] * N_CHUNKS]
        first = True
        for c in range(N_CHUNKS):
            for hv in range(2):
                sbuf_ref[
                    pl.ds(c * cblk, cblk), pl.ds(hv * ncol, ncol)
                ] = jnp.dot(
                    a_ref[pl.ds(send_off[hv] + c * cblk, cblk), :],
                    b_ref[:, hv * ncol : (hv + 1) * ncol],
                    preferred_element_type=jnp.float32,
                ).astype(jnp.bfloat16)
                if first:
                    pl.semaphore_wait(barrier, 4)
                    first = False
                rdma[hv][c] = start_stage(hv, 0, c)

        for hv in range(2):
            out_ref[pl.ds(off[hv], blk), pl.ds(hv * ncol, ncol)] = jnp.dot(
                a_ref[pl.ds(off[hv], blk), :],
                b_ref[:, hv * ncol : (hv + 1) * ncol],
                preferred_element_type=jnp.float32,
            )

        for s in range(4):
            for c in range(N_CHUNKS):
                for hv in range(2):
                    cols = pl.ds(hv * ncol, ncol)
                    orows = pl.ds(off[hv] + c * cblk, cblk)
                    crows = pl.ds(s * blk + c * cblk, cblk)
                    rdma[hv][c].wait_recv()
                    t = (
                        out_ref[orows, cols]
                        + comm_ref[crows, cols].astype(jnp.float32)
                    )
                    if s < 3:
                        out_ref[orows, cols] = t
                        sbuf_ref[
                            pl.ds((s + 1) * blk + c * cblk, cblk), cols
                        ] = t.astype(jnp.bfloat16)
                        rdma[hv][c] = start_stage(hv, s + 1, c)
                    else:
                        partner, _ = _partner_and_bit(my, STAGES[hv][4])
                        gbuf_ref[orows, cols] = t.astype(jnp.bfloat16)
                        r = pltpu.make_async_remote_copy(
                            src_ref=gbuf_ref.at[orows, cols],
                            dst_ref=gbuf_ref.at[orows, cols],
                            send_sem=send_sems.at[hv, 4, c],
                            recv_sem=recv_sems.at[hv, 4, c],
                            device_id=(partner,),
                            device_id_type=pl.DeviceIdType.MESH,
                        )
                        r.start()
                        pending_sends.append(r)
                        rdma[hv][c] = r

        for c in range(N_CHUNKS):
            for hv in range(2):
                rdma[hv][c].wait_recv()

        out_ref[...] = jnp.maximum(gbuf_ref[...].astype(jnp.float32), 0.0)

        for r in pending_sends:
            r.wait_send()

    return pl.pallas_call(
        body,
        out_shape=jax.ShapeDtypeStruct((m, n), jnp.float32),
        in_specs=[
            pl.BlockSpec(memory_space=pltpu.VMEM),
            pl.BlockSpec(memory_space=pltpu.VMEM),
        ],
        out_specs=pl.BlockSpec(memory_space=pltpu.VMEM),
        scratch_shapes=[
            pltpu.VMEM((4 * (m // 2), n), jnp.bfloat16),
            pltpu.VMEM((4 * (m // 2), n), jnp.bfloat16),
            pltpu.VMEM((m, n), jnp.bfloat16),
            pltpu.SemaphoreType.DMA((2, 5, N_CHUNKS)),
            pltpu.SemaphoreType.DMA((2, 5, N_CHUNKS)),
        ],
        compiler_params=pltpu.CompilerParams(collective_id=0),
    )(A, B)
